# Initial kernel scaffold; baseline (speedup 1.0000x reference)
import jax
import jax.numpy as jnp
from jax.experimental import pallas as pl


def kernel(x, edge_index, h_cur, c_cur, W, b):
    raise NotImplementedError("write your pallas kernel here")


# speedup vs baseline: 18.2205x; 18.2205x over previous
import functools

import jax
import jax.numpy as jnp
from jax import lax
from jax.experimental import pallas as pl
from jax.experimental.pallas import tpu as pltpu
from jax.experimental.pallas import tpu_sc as plsc

N = 10000
E = 320000
D = 128
L = 16
NTILES = 32
CH = 128
NCH = 80
NCHH = NCH // 2
EP = NTILES * NCH * CH
NP = 10112
RPT = NP // 16
TCB = 1000
def _deg_body(dst_hbm, ones_hbm, zeros_hbm, deg_out, deg_sh, dstbuf, ones_v):
    c = lax.axis_index("c")
    s = lax.axis_index("s")
    wid = s * 2 + c
    pltpu.sync_copy(ones_hbm, ones_v)
    pltpu.sync_copy(zeros_hbm, deg_sh.at[pl.ds(s * RPT, RPT)])
    plsc.subcore_barrier()

    pltpu.sync_copy(dst_hbm.at[wid], dstbuf)

    def body(k, _):
        pltpu.sync_copy(ones_v, deg_sh.at[dstbuf.at[k]], add=True)
        return 0

    lax.fori_loop(0, NCH, body, 0)
    plsc.subcore_barrier()
    pltpu.sync_copy(deg_sh.at[pl.ds(s * RPT, RPT)],
                    deg_out.at[c, pl.ds(s * RPT, RPT)])


def _edge_body(src_hbm, dst_hbm, y_hbm, zrows_hbm, z_out,
                 z_sh, srcbuf, dstbuf, rbuf0, rbuf1,
                 gsem0, gsem1, ssem0, ssem1):
    c = lax.axis_index("c")
    s = lax.axis_index("s")
    wid = s * 2 + c
    zslice = rbuf0.at[pl.ds(0, RPT // 8)]
    pltpu.sync_copy(zrows_hbm, zslice)
    for t in range(8):
        pltpu.sync_copy(zslice, z_sh.at[pl.ds(s * RPT + t * (RPT // 8), RPT // 8)])
    plsc.subcore_barrier()

    for h in range(2):
        pltpu.sync_copy(src_hbm.at[wid, pl.ds(h * NCHH, NCHH)], srcbuf)
        pltpu.sync_copy(dst_hbm.at[wid, pl.ds(h * NCHH, NCHH)], dstbuf)

        pltpu.async_copy(y_hbm.at[srcbuf.at[0]], rbuf0, gsem0)

        def body(j, _):
            k0 = 2 * j
            k1 = k0 + 1
            pltpu.make_async_copy(y_hbm.at[srcbuf.at[k0]], rbuf0, gsem0).wait()
            sd0 = pltpu.async_copy(rbuf0, z_sh.at[dstbuf.at[k0]], ssem0, add=True)

            @pl.when(j > 0)
            def _():
                pltpu.make_async_copy(rbuf1, z_sh.at[dstbuf.at[k1 - 2]], ssem1).wait()

            pltpu.async_copy(y_hbm.at[srcbuf.at[k1]], rbuf1, gsem1)
            sd0.wait()

            @pl.when(j < NCHH // 2 - 1)
            def _():
                pltpu.async_copy(y_hbm.at[srcbuf.at[k0 + 2]], rbuf0, gsem0)

            pltpu.make_async_copy(y_hbm.at[srcbuf.at[k1]], rbuf1, gsem1).wait()
            pltpu.async_copy(rbuf1, z_sh.at[dstbuf.at[k1]], ssem1, add=True)
            return 0

        lax.fori_loop(0, NCHH // 2, body, 0)
        pltpu.make_async_copy(rbuf1, z_sh.at[dstbuf.at[NCHH - 1]], ssem1).wait()
    plsc.subcore_barrier()
    for t in range(8):
        off = s * RPT + t * (RPT // 8)
        pltpu.sync_copy(z_sh.at[pl.ds(off, RPT // 8)], zslice)
        pltpu.sync_copy(zslice, z_out.at[c, pl.ds(off, RPT // 8)])


@functools.cache
def _build_sc_kernels():
    mesh = plsc.VectorSubcoreMesh(core_axis_name="c", subcore_axis_name="s",
                                  num_cores=2, num_subcores=16)
    params = pltpu.CompilerParams(use_tc_tiling_on_sc=False)
    deg_kernel = pl.kernel(
        _deg_body,
        out_type=jax.ShapeDtypeStruct((2, NP, L), jnp.float32),
        mesh=mesh,
        compiler_params=params,
        scratch_types=[
            pltpu.VMEM_SHARED((NP, L), jnp.float32),
            pltpu.VMEM((NCH, CH), jnp.int32),
            pltpu.VMEM((CH, L), jnp.float32),
        ],
    )
    edge_kernel = pl.kernel(
        _edge_body,
        out_type=jax.ShapeDtypeStruct((2, NP, D), jnp.float32),
        mesh=mesh,
        compiler_params=params,
        scratch_types=[
            pltpu.VMEM_SHARED((NP, D), jnp.float32),
            pltpu.VMEM((NCHH, CH), jnp.int32),
            pltpu.VMEM((NCHH, CH), jnp.int32),
            pltpu.VMEM((CH, D), jnp.float32),
            pltpu.VMEM((CH, D), jnp.float32),
            pltpu.SemaphoreType.DMA,
            pltpu.SemaphoreType.DMA,
            pltpu.SemaphoreType.DMA,
            pltpu.SemaphoreType.DMA,
        ],
    )
    return deg_kernel, edge_kernel


def _tc_prep_body(x_ref, h_ref, w_ref, d0_ref, d1_ref, b_ref, y_ref, base_ref):
    comb = x_ref[...] + h_ref[...]
    xw = lax.dot_general(comb, w_ref[...], (((1,), (0,)), ((), ())),
                         precision=lax.Precision.HIGHEST,
                         preferred_element_type=jnp.float32)
    deg = d0_ref[:, 0:1] + d1_ref[:, 0:1] + 1.0
    dis = lax.rsqrt(deg)
    y_ref[...] = xw * dis
    base_ref[...] = xw * (dis * dis) + b_ref[...]


def _tc_fin_body(z0_ref, z1_ref, d0_ref, d1_ref, base_ref, o_ref):
    deg = d0_ref[:, 0:1] + d1_ref[:, 0:1] + 1.0
    dis = lax.rsqrt(deg)
    o_ref[...] = (z0_ref[...] + z1_ref[...]) * dis + base_ref[...]


def kernel(x, edge_index, h_cur, c_cur, W, b):
    ei = edge_index.astype(jnp.int32)
    pad = EP - E
    srcp = jnp.concatenate([ei[0], jnp.zeros((pad,), jnp.int32)])
    dstp = jnp.concatenate([ei[1], jnp.full((pad,), N, jnp.int32)])
    src3 = srcp.reshape(NTILES, NCH, CH)
    dst3 = dstp.reshape(NTILES, NCH, CH)
    W128 = W[:, :D]
    b128 = b[:D].reshape(1, D)

    deg_kernel, edge_kernel = _build_sc_kernels()
    ones16 = jnp.ones((CH, L), jnp.float32)
    zeros16 = jnp.zeros((RPT, L), jnp.float32)
    zrows = jnp.zeros((RPT // 8, D), jnp.float32)
    deg2 = deg_kernel(dst3, ones16, zeros16)
    d0 = deg2[0]
    d1 = deg2[1]

    row_spec = pl.BlockSpec((TCB, D), lambda i: (i, 0))
    deg_spec = pl.BlockSpec((TCB, L), lambda i: (i, 0))
    y, base = pl.pallas_call(
        _tc_prep_body,
        grid=(N // TCB,),
        in_specs=[
            row_spec,
            row_spec,
            pl.BlockSpec((D, D), lambda i: (0, 0)),
            deg_spec,
            deg_spec,
            pl.BlockSpec((1, D), lambda i: (0, 0)),
        ],
        out_specs=[row_spec, row_spec],
        out_shape=[jax.ShapeDtypeStruct((N, D), jnp.float32)] * 2,
    )(x, h_cur, W128, d0, d1, b128)

    z2 = edge_kernel(src3, dst3, y, zrows)

    out = pl.pallas_call(
        _tc_fin_body,
        grid=(N // TCB,),
        in_specs=[row_spec, row_spec, deg_spec, deg_spec, row_spec],
        out_specs=row_spec,
        out_shape=jax.ShapeDtypeStruct((N, D), jnp.float32),
    )(z2[0], z2[1], d0, d1, base)
    return out


# speedup vs baseline: 38.4666x; 2.1112x over previous
import functools

import jax
import jax.numpy as jnp
from jax import lax
from jax.experimental import pallas as pl
from jax.experimental.pallas import tpu as pltpu
from jax.experimental.pallas import tpu_sc as plsc

N = 10000
E = 320000
D = 128
HD = D // 2
L = 16
NTILES = 32
CH = 128
NCH = 80
EP = NTILES * NCH * CH
NCHT = 162
SEG = NCHT // 2
E2P = 16 * NCHT * CH
NP = 10112
RPT = NP // 16
YPT = N // 16
TCB = 1000


def _deg_body(dst_hbm, ones_hbm, zeros_hbm, deg_out, deg_sh, dstbuf, ones_v):
    c = lax.axis_index("c")
    s = lax.axis_index("s")
    wid = s * 2 + c
    pltpu.sync_copy(ones_hbm, ones_v)
    pltpu.sync_copy(zeros_hbm, deg_sh.at[pl.ds(s * RPT, RPT)])
    plsc.subcore_barrier()

    pltpu.sync_copy(dst_hbm.at[wid], dstbuf)

    def body(k, _):
        pltpu.sync_copy(ones_v, deg_sh.at[dstbuf.at[k]], add=True)
        return 0

    lax.fori_loop(0, NCH, body, 0)
    plsc.subcore_barrier()
    pltpu.sync_copy(deg_sh.at[pl.ds(s * RPT, RPT)],
                    deg_out.at[c, pl.ds(s * RPT, RPT)])


def _edge_body(src_hbm, dst_hbm, ylo_hbm, yhi_hbm, zrows_hbm, z_out,
               z_sh, y_sh, srcbuf, dstbuf, rb0, rb1, rb2,
               g0, g1, g2, t0, t1, t2):
    c = lax.axis_index("c")
    s = lax.axis_index("s")
    rbufs = (rb0, rb1, rb2)
    gsems = (g0, g1, g2)
    ssems = (t0, t1, t2)

    zslice = rb0.at[pl.ds(0, RPT // 8)]
    pltpu.sync_copy(zrows_hbm, zslice)
    for t in range(8):
        pltpu.sync_copy(zslice, z_sh.at[pl.ds(s * RPT + t * (RPT // 8), RPT // 8)])

    ybounce = rb1.at[pl.ds(0, YPT // 5)]

    @pl.when(c == 0)
    def _():
        for t in range(5):
            off = s * YPT + t * (YPT // 5)
            pltpu.sync_copy(ylo_hbm.at[pl.ds(off, YPT // 5)], ybounce)
            pltpu.sync_copy(ybounce, y_sh.at[pl.ds(off, YPT // 5)])

    @pl.when(c == 1)
    def _():
        for t in range(5):
            off = s * YPT + t * (YPT // 5)
            pltpu.sync_copy(yhi_hbm.at[pl.ds(off, YPT // 5)], ybounce)
            pltpu.sync_copy(ybounce, y_sh.at[pl.ds(off, YPT // 5)])

    plsc.subcore_barrier()

    for h in range(2):
        pltpu.sync_copy(src_hbm.at[s, pl.ds(h * SEG, SEG)], srcbuf)
        pltpu.sync_copy(dst_hbm.at[s, pl.ds(h * SEG, SEG)], dstbuf)

        pltpu.async_copy(y_sh.at[srcbuf.at[0]], rbufs[0], gsems[0])
        pltpu.async_copy(y_sh.at[srcbuf.at[1]], rbufs[1], gsems[1])

        def body(i, _):
            for d in range(3):
                k = 3 * i + d
                b = d
                bn = (d + 2) % 3
                if d == 0:
                    @pl.when(i > 0)
                    def _():
                        pltpu.make_async_copy(
                            rbufs[bn], z_sh.at[dstbuf.at[k - 1]], ssems[bn]).wait()

                    pltpu.async_copy(y_sh.at[srcbuf.at[k + 2]], rbufs[bn], gsems[bn])
                else:
                    pltpu.make_async_copy(
                        rbufs[bn], z_sh.at[dstbuf.at[k - 1]], ssems[bn]).wait()

                    @pl.when(i < SEG // 3 - 1)
                    def _():
                        pltpu.async_copy(y_sh.at[srcbuf.at[k + 2]], rbufs[bn], gsems[bn])

                pltpu.make_async_copy(y_sh.at[srcbuf.at[k]], rbufs[b], gsems[b]).wait()
                pltpu.async_copy(rbufs[b], z_sh.at[dstbuf.at[k]], ssems[b], add=True)
            return 0

        lax.fori_loop(0, SEG // 3, body, 0)
        pltpu.make_async_copy(
            rbufs[(SEG - 1) % 3], z_sh.at[dstbuf.at[SEG - 1]],
            ssems[(SEG - 1) % 3]).wait()

    plsc.subcore_barrier()
    for t in range(8):
        off = s * RPT + t * (RPT // 8)
        pltpu.sync_copy(z_sh.at[pl.ds(off, RPT // 8)], zslice)
        pltpu.sync_copy(zslice, z_out.at[c, pl.ds(off, RPT // 8)])


@functools.cache
def _build_sc_kernels():
    mesh = plsc.VectorSubcoreMesh(core_axis_name="c", subcore_axis_name="s",
                                  num_cores=2, num_subcores=16)
    params = pltpu.CompilerParams(use_tc_tiling_on_sc=False)
    deg_kernel = pl.kernel(
        _deg_body,
        out_type=jax.ShapeDtypeStruct((2, NP, L), jnp.float32),
        mesh=mesh,
        compiler_params=params,
        scratch_types=[
            pltpu.VMEM_SHARED((NP, L), jnp.float32),
            pltpu.VMEM((NCH, CH), jnp.int32),
            pltpu.VMEM((CH, L), jnp.float32),
        ],
    )
    edge_kernel = pl.kernel(
        _edge_body,
        out_type=jax.ShapeDtypeStruct((2, NP, HD), jnp.float32),
        mesh=mesh,
        compiler_params=params,
        scratch_types=[
            pltpu.VMEM_SHARED((NP, HD), jnp.float32),
            pltpu.VMEM_SHARED((N, HD), jnp.float32),
            pltpu.VMEM((SEG, CH), jnp.int32),
            pltpu.VMEM((SEG, CH), jnp.int32),
            pltpu.VMEM((CH, HD), jnp.float32),
            pltpu.VMEM((CH, HD), jnp.float32),
            pltpu.VMEM((CH, HD), jnp.float32),
            pltpu.SemaphoreType.DMA,
            pltpu.SemaphoreType.DMA,
            pltpu.SemaphoreType.DMA,
            pltpu.SemaphoreType.DMA,
            pltpu.SemaphoreType.DMA,
            pltpu.SemaphoreType.DMA,
        ],
    )
    return deg_kernel, edge_kernel


def _tc_prep_body(x_ref, h_ref, w_ref, d0_ref, d1_ref, b_ref,
                  ylo_ref, yhi_ref, base_ref):
    comb = x_ref[...] + h_ref[...]
    xw = lax.dot_general(comb, w_ref[...], (((1,), (0,)), ((), ())),
                         precision=lax.Precision.HIGHEST,
                         preferred_element_type=jnp.float32)
    deg = d0_ref[:, 0:1] + d1_ref[:, 0:1] + 1.0
    dis = lax.rsqrt(deg)
    y = xw * dis
    ylo_ref[...] = y[:, :HD]
    yhi_ref[...] = y[:, HD:]
    base_ref[...] = xw * (dis * dis) + b_ref[...]


def _tc_fin_body(z0_ref, z1_ref, d0_ref, d1_ref, base_ref, o_ref):
    deg = d0_ref[:, 0:1] + d1_ref[:, 0:1] + 1.0
    dis = lax.rsqrt(deg)
    z = jnp.concatenate([z0_ref[...], z1_ref[...]], axis=1)
    o_ref[...] = z * dis + base_ref[...]


def kernel(x, edge_index, h_cur, c_cur, W, b):
    ei = edge_index.astype(jnp.int32)
    src = ei[0]
    dst = ei[1]
    dst3 = jnp.concatenate(
        [dst, jnp.full((EP - E,), N, jnp.int32)]).reshape(NTILES, NCH, CH)
    src_e = jnp.concatenate(
        [src, jnp.zeros((E2P - E,), jnp.int32)]).reshape(16, NCHT, CH)
    dst_e = jnp.concatenate(
        [dst, jnp.full((E2P - E,), N, jnp.int32)]).reshape(16, NCHT, CH)
    W128 = W[:, :D]
    b128 = b[:D].reshape(1, D)

    deg_kernel, edge_kernel = _build_sc_kernels()
    ones16 = jnp.ones((CH, L), jnp.float32)
    zeros16 = jnp.zeros((RPT, L), jnp.float32)
    zrows = jnp.zeros((RPT // 8, HD), jnp.float32)
    deg2 = deg_kernel(dst3, ones16, zeros16)
    d0 = deg2[0]
    d1 = deg2[1]

    row_spec = pl.BlockSpec((TCB, D), lambda i: (i, 0))
    half_spec = pl.BlockSpec((TCB, HD), lambda i: (i, 0))
    deg_spec = pl.BlockSpec((TCB, L), lambda i: (i, 0))
    ylo, yhi, base = pl.pallas_call(
        _tc_prep_body,
        grid=(N // TCB,),
        in_specs=[
            row_spec,
            row_spec,
            pl.BlockSpec((D, D), lambda i: (0, 0)),
            deg_spec,
            deg_spec,
            pl.BlockSpec((1, D), lambda i: (0, 0)),
        ],
        out_specs=[half_spec, half_spec, row_spec],
        out_shape=[jax.ShapeDtypeStruct((N, HD), jnp.float32),
                   jax.ShapeDtypeStruct((N, HD), jnp.float32),
                   jax.ShapeDtypeStruct((N, D), jnp.float32)],
    )(x, h_cur, W128, d0, d1, b128)

    z2 = edge_kernel(src_e, dst_e, ylo, yhi, zrows)

    out = pl.pallas_call(
        _tc_fin_body,
        grid=(N // TCB,),
        in_specs=[half_spec, half_spec, deg_spec, deg_spec, row_spec],
        out_specs=row_spec,
        out_shape=jax.ShapeDtypeStruct((N, D), jnp.float32),
    )(z2[0], z2[1], d0, d1, base)
    return out
